# baseline (speedup 1.0000x reference)
import jax
import jax.numpy as jnp
from jax.experimental import pallas as pl

_DN = (((0,), (0,)), ((), ()))


def _mlp_body(f_ref, pm_ref, p_ref, o_ref):
    fpair = f_ref[0]
    ft = fpair.T
    pm = pm_ref[0]
    P = p_ref[...]
    F = ft.shape[0] // 2
    w1 = P[0:67, 0:64]
    w2 = P[72:136, 0:32]
    b1c = P[136:137, 0:64].T
    b2c = P[144:145, 0:32].T
    b3c = P[152:153, 0:1].T
    w3 = P[160:161, 0:32].T

    def half(feats_t, pts_t):
        h = jax.lax.dot_general(w1[:F], feats_t, _DN,
                                preferred_element_type=jnp.float32)
        h = h + jax.lax.dot_general(w1[F:], pts_t, _DN,
                                    preferred_element_type=jnp.float32)
        h = jnp.maximum(h + b1c, 0.0)
        h = jnp.maximum(jax.lax.dot_general(w2, h, _DN,
                                            preferred_element_type=jnp.float32)
                        + b2c, 0.0)
        z = jax.lax.dot_general(w3, h, _DN,
                                preferred_element_type=jnp.float32) + b3c
        return jax.nn.sigmoid(z)

    s_e = half(ft[:F], pm[0:3])
    s_o = half(ft[F:], pm[4:7])
    m_e, m_o = pm[3:4], pm[7:8]
    sc = jnp.concatenate([s_e * m_e, s_o * m_o], axis=0)
    tot = jnp.sum(m_e) + jnp.sum(m_o)
    o_ref[0] = jnp.where(tot < 10.0, jnp.zeros_like(sc), sc)


def kernel(points, features, leaf_mask, W1, b1, W2, b2, W3, b3):
    B, N, F = features.shape
    H = N // 2
    fpair = features.reshape(B, H, 2 * F)
    pm = jnp.concatenate([points, leaf_mask[..., None]], -1) \
        .reshape(B, H, 8).transpose(0, 2, 1)
    P = jnp.concatenate([
        jnp.pad(W1, ((0, 5), (0, 64))),
        jnp.pad(W2, ((0, 0), (0, 96))),
        jnp.pad(b1[None, :], ((0, 7), (0, 64))),
        jnp.pad(b2[None, :], ((0, 7), (0, 96))),
        jnp.pad(b3[None, :], ((0, 7), (0, 127))),
        jnp.pad(W3.T, ((0, 15), (0, 96))),
    ], axis=0)

    out = pl.pallas_call(
        _mlp_body,
        grid=(B,),
        in_specs=[
            pl.BlockSpec((1, H, 2 * F), lambda b: (b, 0, 0)),
            pl.BlockSpec((1, 8, H), lambda b: (b, 0, 0)),
            pl.BlockSpec(P.shape, lambda b: (0, 0)),
        ],
        out_specs=pl.BlockSpec((1, 2, H), lambda b: (b, 0, 0)),
        out_shape=jax.ShapeDtypeStruct((B, 2, H), jnp.float32),
    )(fpair, pm, P)
    return out.transpose(0, 2, 1).reshape(B, N)
